# baseline (speedup 1.0000x reference)
import functools

import jax
import jax.numpy as jnp
from jax import lax
from jax.experimental import pallas as pl
from jax.experimental.pallas import tpu as pltpu
from jax.experimental.pallas import tpu_sc as plsc

N = 10000
E = 320000
D = 128
DE = 16
L = 4
C = 4
BN_EPS = 1e-5

NC = 2
NS = 16
NW = NC * NS
EPW = E // NW
B = 40
NBLK = EPW // B
NPT = N // NS
SL = D // 16


def _edge_lin_body(ea_ref, wf_ref, bf_ref, out_ref):
    out_ref[0] = (
        jnp.dot(ea_ref[...], wf_ref[0], preferred_element_type=jnp.float32)
        + bf_ref[0, 0][None, :]
    )


def _edge_lin(edge_attr, wf, bf, eb=4000):
    return pl.pallas_call(
        _edge_lin_body,
        grid=(L, E // eb),
        in_specs=[
            pl.BlockSpec((eb, DE), lambda l, e: (e, 0)),
            pl.BlockSpec((1, DE, D), lambda l, e: (l, 0, 0)),
            pl.BlockSpec((1, 1, D), lambda l, e: (l, 0, 0)),
        ],
        out_specs=pl.BlockSpec((1, eb, D), lambda l, e: (l, e, 0)),
        out_shape=jax.ShapeDtypeStruct((L, E, D), jnp.float32),
    )(edge_attr, wf, bf)


def _sc_layer_body(layer, h_hbm, eib_hbm, c_hbm, out_hbm,
                   ibuf, rows, cv, mv, agg,
                   si0, si1, si2, si3, sg0, sg1, sc0, sc1, ss0, ss1):
    cid = lax.axis_index("c")
    sid = lax.axis_index("s")
    wid = cid * NS + sid
    si = (si0, si1, si2, si3)
    sg = (sg0, sg1)
    sc = (sc0, sc1)
    ss = (ss0, ss1)

    def idx_start(blk, q):
        pltpu.async_copy(eib_hbm.at[wid, blk], ibuf.at[q], si[q])

    def idx_wait(blk, q):
        pltpu.make_async_copy(eib_hbm.at[wid, blk], ibuf.at[q], si[q]).wait()

    def fetch_start(blk, s, q):
        pltpu.async_copy(h_hbm.at[ibuf.at[q, 0]], rows.at[s], sg[s])
        pltpu.async_copy(c_hbm.at[layer, pl.ds(wid * EPW + blk * B, B)],
                         cv.at[s], sc[s])

    def fetch_wait(blk, s, q):
        pltpu.make_async_copy(h_hbm.at[ibuf.at[q, 0]], rows.at[s],
                              sg[s]).wait()
        pltpu.make_async_copy(c_hbm.at[layer, pl.ds(wid * EPW + blk * B, B)],
                              cv.at[s], sc[s]).wait()

    def compute(s):
        r_ref, c_ref, m_ref = rows.at[s], cv.at[s], mv.at[s]

        @plsc.parallel_loop(0, B, 1, unroll=4)
        def _(e):
            for j in range(SL):
                sl = pl.ds(j * 16, 16)
                m_ref[e, sl] = jnp.maximum(r_ref[e, sl] + c_ref[e, sl], 0.0)

    def scat_start(blk, s, q):
        pltpu.async_copy(mv.at[s], agg.at[ibuf.at[q, 1]], ss[s], add=True)

    def scat_wait(blk, s, q):
        pltpu.make_async_copy(mv.at[s], agg.at[ibuf.at[q, 1]],
                              ss[s]).wait()

    idx_start(0, 0)
    idx_start(1, 1)

    z0 = mv.at[0]

    @plsc.parallel_loop(0, B, 1, unroll=4)
    def _(i):
        for j in range(SL):
            z0[i, pl.ds(j * 16, 16)] = jnp.zeros((16,), jnp.float32)
    row0 = sid * 624
    for r in range(15):
        pltpu.sync_copy(z0, agg.at[pl.ds(row0 + r * B, B)])
    pltpu.sync_copy(z0.at[pl.ds(0, 24)], agg.at[pl.ds(row0 + 600, 24)])

    @pl.when(sid == NS - 1)
    def _():
        pltpu.sync_copy(z0.at[pl.ds(0, 16)], agg.at[pl.ds(N - 16, 16)])

    plsc.subcore_barrier()

    idx_wait(0, 0)
    fetch_start(0, 0, 0)

    def quad_body(i, _):
        for u in range(4):
            blk = 4 * i + u
            s = u % 2
            fetch_wait(blk, s, u)
            if u < 2:
                @pl.when(i >= 1)
                def _():
                    scat_wait(blk - 2, s, (u + 2) % 4)
            else:
                scat_wait(blk - 2, s, (u + 2) % 4)
            idx_start(blk + 2, (u + 2) % 4)
            compute(s)
            scat_start(blk, s, u)
            idx_wait(blk + 1, (u + 1) % 4)
            fetch_start(blk + 1, 1 - s, (u + 1) % 4)
        return 0

    lax.fori_loop(0, (NBLK - 2) // 4, quad_body, 0)

    tb = NBLK - 2
    fetch_wait(tb, 0, 0)
    scat_wait(tb - 2, 0, 2)
    compute(0)
    scat_start(tb, 0, 0)
    idx_wait(tb + 1, 1)
    fetch_start(tb + 1, 1, 1)
    fetch_wait(tb + 1, 1, 1)
    scat_wait(tb - 1, 1, 3)
    compute(1)
    scat_start(tb + 1, 1, 1)
    scat_wait(tb, 0, 0)
    scat_wait(tb + 1, 1, 1)

    plsc.subcore_barrier()

    for r in range(15):
        pltpu.sync_copy(agg.at[pl.ds(row0 + r * B, B)],
                        out_hbm.at[cid, pl.ds(row0 + r * B, B)])
    pltpu.sync_copy(agg.at[pl.ds(row0 + 600, 24)],
                    out_hbm.at[cid, pl.ds(row0 + 600, 24)])

    @pl.when(sid == NS - 1)
    def _():
        pltpu.sync_copy(agg.at[pl.ds(N - 16, 16)],
                        out_hbm.at[cid, pl.ds(N - 16, 16)])


def _sc_layer(layer, h, eib, c_all):
    mesh = plsc.VectorSubcoreMesh(core_axis_name="c", subcore_axis_name="s")
    kern = functools.partial(
        pl.kernel,
        out_type=jax.ShapeDtypeStruct((NC, N, D), jnp.float32),
        mesh=mesh,
        scratch_types=[
            pltpu.VMEM((4, 2, B), jnp.int32),
            pltpu.VMEM((2, B, D), jnp.float32),
            pltpu.VMEM((2, B, D), jnp.float32),
            pltpu.VMEM((2, B, D), jnp.float32),
            pltpu.VMEM_SHARED((N, D), jnp.float32),
        ] + [pltpu.SemaphoreType.DMA] * 10,
    )(functools.partial(_sc_layer_body, layer))
    return kern(h, eib, c_all)


def _mlp_body(first, w_out, h_ref, p_ref, w1_ref, b1_ref, w2_ref, b2_ref,
              gs_ref, beta_ref, wo_ref, out_ref):
    h = h_ref[...]
    z = h + p_ref[0] + p_ref[1]
    t = jnp.maximum(jnp.dot(z, w1_ref[...], preferred_element_type=jnp.float32)
                    + b1_ref[0][None, :], 0.0)
    u = jnp.dot(t, w2_ref[...], preferred_element_type=jnp.float32) + b2_ref[0][None, :]
    v = gs_ref[0][None, :] * u + beta_ref[0][None, :]
    r = jnp.maximum(v, 0.0)
    hn = r if first else h + r
    if w_out:
        out_ref[...] = jnp.dot(hn, wo_ref[...], preferred_element_type=jnp.float32)
    else:
        out_ref[...] = hn


def _mlp(h, parts, w1, b1, w2, b2, gs, beta, wo, first, w_out, nb=2000):
    return pl.pallas_call(
        functools.partial(_mlp_body, first, w_out),
        grid=(N // nb,),
        in_specs=[
            pl.BlockSpec((nb, D), lambda i: (i, 0)),
            pl.BlockSpec((NC, nb, D), lambda i: (0, i, 0)),
            pl.BlockSpec((D, D), lambda i: (0, 0)),
            pl.BlockSpec((1, D), lambda i: (0, 0)),
            pl.BlockSpec((D, D), lambda i: (0, 0)),
            pl.BlockSpec((1, D), lambda i: (0, 0)),
            pl.BlockSpec((1, D), lambda i: (0, 0)),
            pl.BlockSpec((1, D), lambda i: (0, 0)),
            pl.BlockSpec((D, D), lambda i: (0, 0)),
        ],
        out_specs=pl.BlockSpec((nb, D), lambda i: (i, 0)),
        out_shape=jax.ShapeDtypeStruct((N, D), jnp.float32),
    )(h, parts, w1, b1, w2, b2, gs, beta, wo)


def kernel(x, edge_index, edge_attr, W_ee, b_ee, Wlin, blin, W1, b1, W2, b2,
           gamma, beta, Wc, bc):
    wf = jnp.einsum("ed,ldo->leo", W_ee, Wlin)
    bf = b_ee @ Wlin + blin
    gs = gamma / jnp.sqrt(1.0 + BN_EPS)
    wc_pad = jnp.zeros((D, D), jnp.float32).at[:, :C].set(Wc)

    eib = edge_index.reshape(2, NW, NBLK, B).transpose(1, 2, 0, 3)

    c_all = _edge_lin(edge_attr, wf, bf[:, None, :])

    h = x
    for l in range(L):
        parts = _sc_layer(l, h, eib, c_all)
        h = _mlp(h, parts,
                 W1[l], b1[l][None], W2[l], b2[l][None],
                 gs[l][None], beta[l][None], wc_pad,
                 first=(l == 0), w_out=(l == L - 1))

    out = h[:, :C] + bc[None, :]
    return out


# speedup vs baseline: 1.3370x; 1.3370x over previous
import functools

import jax
import jax.numpy as jnp
from jax import lax
from jax.experimental import pallas as pl
from jax.experimental.pallas import tpu as pltpu
from jax.experimental.pallas import tpu_sc as plsc

N = 10000
E = 320000
D = 128
DE = 16
L = 4
C = 4
BN_EPS = 1e-5

NC = 2
NS = 16
NW = NC * NS
EPW = E // NW
B = 40
NBLK = EPW // B
NPT = N // NS
SL = D // 16


def _edge_lin_body(ea_ref, wf_ref, bf_ref, out_ref):
    out_ref[...] = (
        jnp.dot(ea_ref[...], wf_ref[...], preferred_element_type=jnp.float32)
        + bf_ref[0][None, :]
    )


def _edge_lin(edge_attr, wf_l, bf_l, eb=8000):
    return pl.pallas_call(
        _edge_lin_body,
        grid=(E // eb,),
        in_specs=[
            pl.BlockSpec((eb, DE), lambda e: (e, 0)),
            pl.BlockSpec((DE, D), lambda e: (0, 0)),
            pl.BlockSpec((1, D), lambda e: (0, 0)),
        ],
        out_specs=pl.BlockSpec((eb, D), lambda e: (e, 0)),
        out_shape=jax.ShapeDtypeStruct((E, D), jnp.float32),
    )(edge_attr, wf_l, bf_l)


def _sc_layer_body(h_hbm, eib_hbm, c_hbm, out_hbm,
                   ibuf, rows, cv, mv, agg,
                   si0, si1, si2, si3, sg0, sg1, sc0, sc1, ss0, ss1):
    cid = lax.axis_index("c")
    sid = lax.axis_index("s")
    wid = cid * NS + sid
    si = (si0, si1, si2, si3)
    sg = (sg0, sg1)
    sc = (sc0, sc1)
    ss = (ss0, ss1)

    def idx_start(blk, q):
        pltpu.async_copy(eib_hbm.at[wid, blk], ibuf.at[q], si[q])

    def idx_wait(blk, q):
        pltpu.make_async_copy(eib_hbm.at[wid, blk], ibuf.at[q], si[q]).wait()

    def fetch_start(blk, s, q):
        pltpu.async_copy(h_hbm.at[ibuf.at[q, 0]], rows.at[s], sg[s])
        pltpu.async_copy(c_hbm.at[pl.ds(wid * EPW + blk * B, B)],
                         cv.at[s], sc[s])

    def fetch_wait(blk, s, q):
        pltpu.make_async_copy(h_hbm.at[ibuf.at[q, 0]], rows.at[s],
                              sg[s]).wait()
        pltpu.make_async_copy(c_hbm.at[pl.ds(wid * EPW + blk * B, B)],
                              cv.at[s], sc[s]).wait()

    def compute(s):
        r_ref, c_ref, m_ref = rows.at[s], cv.at[s], mv.at[s]

        @plsc.parallel_loop(0, B, 1, unroll=4)
        def _(e):
            for j in range(SL):
                sl = pl.ds(j * 16, 16)
                m_ref[e, sl] = jnp.maximum(r_ref[e, sl] + c_ref[e, sl], 0.0)

    def scat_start(blk, s, q):
        pltpu.async_copy(mv.at[s], agg.at[ibuf.at[q, 1]], ss[s], add=True)

    def scat_wait(blk, s, q):
        pltpu.make_async_copy(mv.at[s], agg.at[ibuf.at[q, 1]],
                              ss[s]).wait()

    idx_start(0, 0)
    idx_start(1, 1)

    z0 = mv.at[0]

    @plsc.parallel_loop(0, B, 1, unroll=4)
    def _(i):
        for j in range(SL):
            z0[i, pl.ds(j * 16, 16)] = jnp.zeros((16,), jnp.float32)
    row0 = sid * 624
    for r in range(15):
        pltpu.sync_copy(z0, agg.at[pl.ds(row0 + r * B, B)])
    pltpu.sync_copy(z0.at[pl.ds(0, 24)], agg.at[pl.ds(row0 + 600, 24)])

    @pl.when(sid == NS - 1)
    def _():
        pltpu.sync_copy(z0.at[pl.ds(0, 16)], agg.at[pl.ds(N - 16, 16)])

    plsc.subcore_barrier()

    idx_wait(0, 0)
    fetch_start(0, 0, 0)

    def quad_body(i, _):
        for u in range(4):
            blk = 4 * i + u
            s = u % 2
            fetch_wait(blk, s, u)
            idx_wait(blk + 1, (u + 1) % 4)
            fetch_start(blk + 1, 1 - s, (u + 1) % 4)
            if u < 2:
                @pl.when(i >= 1)
                def _():
                    scat_wait(blk - 2, s, (u + 2) % 4)
            else:
                scat_wait(blk - 2, s, (u + 2) % 4)
            idx_start(blk + 2, (u + 2) % 4)
            compute(s)
            scat_start(blk, s, u)
        return 0

    lax.fori_loop(0, (NBLK - 2) // 4, quad_body, 0)

    tb = NBLK - 2
    fetch_wait(tb, 0, 0)
    idx_wait(tb + 1, 1)
    fetch_start(tb + 1, 1, 1)
    scat_wait(tb - 2, 0, 2)
    compute(0)
    scat_start(tb, 0, 0)
    fetch_wait(tb + 1, 1, 1)
    scat_wait(tb - 1, 1, 3)
    compute(1)
    scat_start(tb + 1, 1, 1)
    scat_wait(tb, 0, 0)
    scat_wait(tb + 1, 1, 1)

    plsc.subcore_barrier()

    for r in range(15):
        pltpu.sync_copy(agg.at[pl.ds(row0 + r * B, B)],
                        out_hbm.at[cid, pl.ds(row0 + r * B, B)])
    pltpu.sync_copy(agg.at[pl.ds(row0 + 600, 24)],
                    out_hbm.at[cid, pl.ds(row0 + 600, 24)])

    @pl.when(sid == NS - 1)
    def _():
        pltpu.sync_copy(agg.at[pl.ds(N - 16, 16)],
                        out_hbm.at[cid, pl.ds(N - 16, 16)])


def _sc_layer(h, eib, c_l):
    mesh = plsc.VectorSubcoreMesh(core_axis_name="c", subcore_axis_name="s")
    kern = functools.partial(
        pl.kernel,
        out_type=jax.ShapeDtypeStruct((NC, N, D), jnp.float32),
        mesh=mesh,
        scratch_types=[
            pltpu.VMEM((4, 2, B), jnp.int32),
            pltpu.VMEM((2, B, D), jnp.float32),
            pltpu.VMEM((2, B, D), jnp.float32),
            pltpu.VMEM((2, B, D), jnp.float32),
            pltpu.VMEM_SHARED((N, D), jnp.float32),
        ] + [pltpu.SemaphoreType.DMA] * 10,
    )(_sc_layer_body)
    return kern(h, eib, c_l)


def _mlp_body(first, w_out, h_ref, p_ref, w1_ref, b1_ref, w2_ref, b2_ref,
              gs_ref, beta_ref, wo_ref, out_ref):
    h = h_ref[...]
    z = h + p_ref[0] + p_ref[1]
    t = jnp.maximum(jnp.dot(z, w1_ref[...], preferred_element_type=jnp.float32)
                    + b1_ref[0][None, :], 0.0)
    u = jnp.dot(t, w2_ref[...], preferred_element_type=jnp.float32) + b2_ref[0][None, :]
    v = gs_ref[0][None, :] * u + beta_ref[0][None, :]
    r = jnp.maximum(v, 0.0)
    hn = r if first else h + r
    if w_out:
        out_ref[...] = jnp.dot(hn, wo_ref[...], preferred_element_type=jnp.float32)
    else:
        out_ref[...] = hn


def _mlp(h, parts, w1, b1, w2, b2, gs, beta, wo, first, w_out, nb=2000):
    return pl.pallas_call(
        functools.partial(_mlp_body, first, w_out),
        grid=(N // nb,),
        in_specs=[
            pl.BlockSpec((nb, D), lambda i: (i, 0)),
            pl.BlockSpec((NC, nb, D), lambda i: (0, i, 0)),
            pl.BlockSpec((D, D), lambda i: (0, 0)),
            pl.BlockSpec((1, D), lambda i: (0, 0)),
            pl.BlockSpec((D, D), lambda i: (0, 0)),
            pl.BlockSpec((1, D), lambda i: (0, 0)),
            pl.BlockSpec((1, D), lambda i: (0, 0)),
            pl.BlockSpec((1, D), lambda i: (0, 0)),
            pl.BlockSpec((D, D), lambda i: (0, 0)),
        ],
        out_specs=pl.BlockSpec((nb, D), lambda i: (i, 0)),
        out_shape=jax.ShapeDtypeStruct((N, D), jnp.float32),
    )(h, parts, w1, b1, w2, b2, gs, beta, wo)


def kernel(x, edge_index, edge_attr, W_ee, b_ee, Wlin, blin, W1, b1, W2, b2,
           gamma, beta, Wc, bc):
    wf = jnp.einsum("ed,ldo->leo", W_ee, Wlin)
    bf = b_ee @ Wlin + blin
    gs = gamma / jnp.sqrt(1.0 + BN_EPS)
    wc_pad = jnp.zeros((D, D), jnp.float32).at[:, :C].set(Wc)

    eib = edge_index.reshape(2, NW, NBLK, B).transpose(1, 2, 0, 3)

    cs = [_edge_lin(edge_attr, wf[l], bf[l][None]) for l in range(L)]

    h = x
    for l in range(L):
        parts = _sc_layer(h, eib, cs[l])
        h = _mlp(h, parts,
                 W1[l], b1[l][None], W2[l], b2[l][None],
                 gs[l][None], beta[l][None], wc_pad,
                 first=(l == 0), w_out=(l == L - 1))

    out = h[:, :C] + bc[None, :]
    return out
